# baseline (speedup 1.0000x reference)
import functools

import jax
import jax.numpy as jnp
from jax import lax
from jax.experimental import pallas as pl
from jax.experimental.pallas import tpu as pltpu
from jax.experimental.pallas import tpu_sc as plsc

B = 4
N = 4096
NPOINT = 1024
K = 16
IN_DIM = 128
OUT_DIM = 256
EPS = 1e-5
CP = 144
ROWS = B * NPOINT * K
NW = 32
RPW = ROWS // NW
CH = 128
NCH = RPW // CH
R = 2048
QB = R // K
_TPB = NPOINT * K // R
TQ = 256


def _fps_body(xyz12_ref, nxc_ref):
    xyz12 = xyz12_ref[...]
    x = xyz12[0:B, :]
    y = xyz12[B:2 * B, :]
    z = xyz12[2 * B:3 * B, :]
    lane = lax.broadcasted_iota(jnp.int32, (B, N), 1)
    lane12 = lax.broadcasted_iota(jnp.int32, (3 * B, N), 1)
    lp_c = lax.broadcasted_iota(jnp.int32, (B, 3, NPOINT), 2)

    def step(i, carry):
        dists, far, nxc = carry
        far12 = jnp.concatenate([far, far, far], axis=0)
        c12 = jnp.sum(jnp.where(lane12 == far12, xyz12, 0.0),
                      axis=1, keepdims=True)
        cx = c12[0:B]
        cy = c12[B:2 * B]
        cz = c12[2 * B:3 * B]
        c3 = jnp.concatenate([cx, cy, cz], axis=1)
        nxc = jnp.where(lp_c == i, c3[:, :, None], nxc)
        dx = x - cx
        dy = y - cy
        dz = z - cz
        d = (dx * dx + dy * dy) + dz * dz
        dists = jnp.minimum(dists, d)
        m = jnp.max(dists, axis=1, keepdims=True)
        far = jnp.min(jnp.where(dists == m, lane, N), axis=1, keepdims=True)
        return dists, far, nxc

    dists0 = jnp.full((B, N), 1e10, jnp.float32)
    far0 = jnp.zeros((B, 1), jnp.int32)
    nxc0 = jnp.zeros((B, 3, NPOINT), jnp.float32)
    _, _, nxc = lax.fori_loop(0, NPOINT, step, (dists0, far0, nxc0))
    nxc_ref[...] = nxc


def _run_fps(xyz):
    xyz12 = jnp.transpose(xyz, (1, 0, 2)).reshape(3 * B, N)
    nxc = pl.pallas_call(
        _fps_body,
        out_shape=jax.ShapeDtypeStruct((B, 3, NPOINT), jnp.float32),
    )(xyz12)
    return nxc, jnp.transpose(nxc, (0, 2, 1))


def _knn_body(nxr_ref, xyz_ref, idx_ref):
    b = pl.program_id(0)
    q = nxr_ref[0]
    a = xyz_ref[0]
    x = a[0:1, :]
    y = a[1:2, :]
    z = a[2:3, :]
    dx = q[:, 0:1] - x
    dy = q[:, 1:2] - y
    dz = q[:, 2:3] - z
    d2 = (dx * dx + dy * dy) + dz * dz
    lane = lax.broadcasted_iota(jnp.int32, (TQ, N), 1)
    base = b * N
    ams = []
    for k in range(K):
        m = jnp.min(d2, axis=1, keepdims=True)
        am = jnp.min(jnp.where(d2 == m, lane, N), axis=1, keepdims=True)
        ams.append(am)
        d2 = jnp.where(lane == am, jnp.float32(jnp.inf), d2)
    idx_ref[0] = jnp.concatenate(ams, axis=1) + base


def _run_knn(nxr, xyz):
    return pl.pallas_call(
        _knn_body,
        grid=(B, NPOINT // TQ),
        in_specs=[
            pl.BlockSpec((1, TQ, 3), lambda b, t: (b, t, 0)),
            pl.BlockSpec((1, 3, N), lambda b, t: (b, 0, 0)),
        ],
        out_specs=pl.BlockSpec((1, TQ, K), lambda b, t: (b, t, 0)),
        out_shape=jax.ShapeDtypeStruct((B, NPOINT, K), jnp.int32),
    )(nxr, xyz)


NT = 512


def _format_body(xyz_ref, pts_ref, tab_ref):
    x3 = xyz_ref[0]
    p = pts_ref[0]
    pad = jnp.zeros((NT, CP - 3 - IN_DIM), jnp.float32)
    tab_ref[...] = jnp.concatenate([x3.T, p.T, pad], axis=1)


def _run_format(xyz, points):
    return pl.pallas_call(
        _format_body,
        grid=(B, N // NT),
        in_specs=[
            pl.BlockSpec((1, 3, NT), lambda b, t: (b, 0, t)),
            pl.BlockSpec((1, IN_DIM, NT), lambda b, t: (b, 0, t)),
        ],
        out_specs=pl.BlockSpec(
            (NT, CP), lambda b, t: (b * (N // NT) + t, 0)),
        out_shape=jax.ShapeDtypeStruct((B * N, CP), jnp.float32),
    )(xyz, points)


def _sc_gather_body(table_hbm, idx_hbm, out_hbm, idx_v, buf0, buf1, sem0, sem1):
    c = lax.axis_index("c")
    s = lax.axis_index("s")
    wid = s * 2 + c
    pltpu.sync_copy(idx_hbm.at[pl.ds(wid * NCH, NCH)], idx_v)
    bufs = (buf0, buf1)
    sems = (sem0, sem1)
    descs = [None, None]
    descs[0] = pltpu.async_copy(table_hbm.at[idx_v.at[0]], buf0, sem0)
    for j in range(NCH):
        cur = j % 2
        if j + 1 < NCH:
            nxt = (j + 1) % 2
            descs[nxt] = pltpu.async_copy(
                table_hbm.at[idx_v.at[j + 1]], bufs[nxt], sems[nxt])
        descs[cur].wait()
        pltpu.sync_copy(bufs[cur], out_hbm.at[pl.ds(wid * RPW + j * CH, CH)])


@functools.cache
def _sc_gather_call():
    mesh = plsc.VectorSubcoreMesh(
        core_axis_name="c", subcore_axis_name="s", num_cores=2, num_subcores=16)
    return pl.kernel(
        _sc_gather_body,
        out_type=jax.ShapeDtypeStruct((ROWS, CP), jnp.float32),
        mesh=mesh,
        scratch_types=[
            pltpu.VMEM((NCH, CH), jnp.int32),
            pltpu.VMEM((CH, CP), jnp.float32),
            pltpu.VMEM((CH, CP), jnp.float32),
            pltpu.SemaphoreType.DMA,
            pltpu.SemaphoreType.DMA,
        ],
        compiler_params=pltpu.CompilerParams(use_tc_tiling_on_sc=False),
    )


def _gather_rows(table, idxf):
    return _sc_gather_call()(table, idxf.reshape(ROWS // CH, CH))


def _passA_body(g_ref, nx_ref, w1_ref, w1x_ref, b1_ref, y1_ref, gn_ref, st_ref):
    i = pl.program_id(0)
    g = g_ref[...]
    nx = nx_ref[...]
    nxe = jnp.broadcast_to(nx[:, None, :], (QB, K, 3)).reshape(R, 3)
    gn = g[:, 0:3] - nxe
    gn_ref[0] = gn.T.reshape(3, QB, K)
    y = jnp.dot(g, w1_ref[...], preferred_element_type=jnp.float32)
    corr = jnp.dot(nx, w1x_ref[...], preferred_element_type=jnp.float32)
    corre = jnp.broadcast_to(
        corr[:, None, :], (QB, K, IN_DIM)).reshape(R, IN_DIM)
    y = y - corre + b1_ref[...]
    y1_ref[...] = y

    @pl.when(i == 0)
    def _():
        st_ref[...] = jnp.zeros_like(st_ref)

    st_ref[0:1, :] += jnp.sum(y, axis=0, keepdims=True)
    st_ref[1:2, :] += jnp.sum(y * y, axis=0, keepdims=True)


def _run_passA(g, nxq, w1t, w1xt, b1r):
    return pl.pallas_call(
        _passA_body,
        grid=(ROWS // R,),
        in_specs=[
            pl.BlockSpec((R, CP), lambda i: (i, 0)),
            pl.BlockSpec((QB, 3), lambda i: (i, 0)),
            pl.BlockSpec((CP, IN_DIM), lambda i: (0, 0)),
            pl.BlockSpec((3, IN_DIM), lambda i: (0, 0)),
            pl.BlockSpec((1, IN_DIM), lambda i: (0, 0)),
        ],
        out_specs=(
            pl.BlockSpec((R, IN_DIM), lambda i: (i, 0)),
            pl.BlockSpec((1, 3, QB, K), lambda i: (i // _TPB, 0, i % _TPB, 0)),
            pl.BlockSpec((8, IN_DIM), lambda i: (0, 0)),
        ),
        out_shape=(
            jax.ShapeDtypeStruct((ROWS, IN_DIM), jnp.float32),
            jax.ShapeDtypeStruct((B, 3, NPOINT, K), jnp.float32),
            jax.ShapeDtypeStruct((8, IN_DIM), jnp.float32),
        ),
    )(g, nxq, w1t, w1xt, b1r)


def _bn_coeffs(st_ref, gamma_ref, beta_ref):
    mean = st_ref[0:1, :] * (1.0 / ROWS)
    var = st_ref[1:2, :] * (1.0 / ROWS) - mean * mean
    a = gamma_ref[...] * lax.rsqrt(var + EPS)
    c = beta_ref[...] - mean * a
    return a, c


def _passB_body(y1_ref, st_ref, g1_ref, be1_ref, w2_ref, b2_ref, y2_ref, st2_ref):
    i = pl.program_id(0)
    a, c = _bn_coeffs(st_ref, g1_ref, be1_ref)
    h = jnp.maximum(y1_ref[...] * a + c, 0.0)
    y = jnp.dot(h, w2_ref[...], preferred_element_type=jnp.float32)
    y = y + b2_ref[...]
    y2_ref[...] = y

    @pl.when(i == 0)
    def _():
        st2_ref[...] = jnp.zeros_like(st2_ref)

    st2_ref[0:1, :] += jnp.sum(y, axis=0, keepdims=True)
    st2_ref[1:2, :] += jnp.sum(y * y, axis=0, keepdims=True)


def _run_passB(y1, st1, g1r, be1r, w2t, b2r):
    return pl.pallas_call(
        _passB_body,
        grid=(ROWS // R,),
        in_specs=[
            pl.BlockSpec((R, IN_DIM), lambda i: (i, 0)),
            pl.BlockSpec((8, IN_DIM), lambda i: (0, 0)),
            pl.BlockSpec((1, IN_DIM), lambda i: (0, 0)),
            pl.BlockSpec((1, IN_DIM), lambda i: (0, 0)),
            pl.BlockSpec((IN_DIM, OUT_DIM), lambda i: (0, 0)),
            pl.BlockSpec((1, OUT_DIM), lambda i: (0, 0)),
        ],
        out_specs=(
            pl.BlockSpec((R, OUT_DIM), lambda i: (i, 0)),
            pl.BlockSpec((8, OUT_DIM), lambda i: (0, 0)),
        ),
        out_shape=(
            jax.ShapeDtypeStruct((ROWS, OUT_DIM), jnp.float32),
            jax.ShapeDtypeStruct((8, OUT_DIM), jnp.float32),
        ),
    )(y1, st1, g1r, be1r, w2t, b2r)


def _passC_body(y2_ref, st_ref, g2_ref, be2_ref, np_ref, pool_ref):
    a, c = _bn_coeffs(st_ref, g2_ref, be2_ref)
    z = jnp.maximum(y2_ref[...] * a + c, 0.0)
    np_ref[0] = z.T.reshape(OUT_DIM, QB, K)
    pool_ref[0] = jnp.max(z.reshape(QB, K, OUT_DIM), axis=1).T


def _run_passC(y2, st2, g2r, be2r):
    return pl.pallas_call(
        _passC_body,
        grid=(ROWS // R,),
        in_specs=[
            pl.BlockSpec((R, OUT_DIM), lambda i: (i, 0)),
            pl.BlockSpec((8, OUT_DIM), lambda i: (0, 0)),
            pl.BlockSpec((1, OUT_DIM), lambda i: (0, 0)),
            pl.BlockSpec((1, OUT_DIM), lambda i: (0, 0)),
        ],
        out_specs=(
            pl.BlockSpec((1, OUT_DIM, QB, K),
                         lambda i: (i // _TPB, 0, i % _TPB, 0)),
            pl.BlockSpec((1, OUT_DIM, QB), lambda i: (i // _TPB, 0, i % _TPB)),
        ),
        out_shape=(
            jax.ShapeDtypeStruct((B, OUT_DIM, NPOINT, K), jnp.float32),
            jax.ShapeDtypeStruct((B, OUT_DIM, NPOINT), jnp.float32),
        ),
    )(y2, st2, g2r, be2r)


def kernel(xyz, points, W1, b1, gamma1, beta1, W2, b2, gamma2, beta2):
    table = _run_format(xyz, points)

    nxc, nxr = _run_fps(xyz)
    idxf = _run_knn(nxr, xyz)
    g = _gather_rows(table, idxf.reshape(-1))

    nxq = nxr.reshape(B * NPOINT, 3)
    w1t = jnp.pad(W1.T, ((0, CP - (IN_DIM + 3)), (0, 0)))
    w1xt = W1[:, 0:3].T
    y1, gn, st1 = _run_passA(
        g, nxq, w1t, w1xt, b1.reshape(1, IN_DIM))

    y2, st2 = _run_passB(
        y1, st1, gamma1.reshape(1, IN_DIM), beta1.reshape(1, IN_DIM),
        W2.T, b2.reshape(1, OUT_DIM))

    np4, pool = _run_passC(
        y2, st2, gamma2.reshape(1, OUT_DIM), beta2.reshape(1, OUT_DIM))

    return (nxc, pool, gn, np4)


# speedup vs baseline: 1.2721x; 1.2721x over previous
import functools

import jax
import jax.numpy as jnp
from jax import lax
from jax.experimental import pallas as pl
from jax.experimental.pallas import tpu as pltpu
from jax.experimental.pallas import tpu_sc as plsc

B = 4
N = 4096
NPOINT = 1024
K = 16
IN_DIM = 128
OUT_DIM = 256
EPS = 1e-5
CP = 144
ROWS = B * NPOINT * K
NW = 32
RPW = ROWS // NW
CH = 128
NCH = RPW // CH
R = 2048
QB = R // K
_TPB = NPOINT * K // R
TQ = 256


def _fps_body(xyz12_ref, nxc_ref):
    xyz12 = xyz12_ref[...]
    x = xyz12[0:B, :]
    y = xyz12[B:2 * B, :]
    z = xyz12[2 * B:3 * B, :]
    lane = lax.broadcasted_iota(jnp.int32, (B, N), 1)
    lane12 = lax.broadcasted_iota(jnp.int32, (3 * B, N), 1)
    lp_c = lax.broadcasted_iota(jnp.int32, (B, 3, NPOINT), 2)

    def step(i, carry):
        dists, far, nxc = carry
        far12 = jnp.concatenate([far, far, far], axis=0)
        c12 = jnp.sum(jnp.where(lane12 == far12, xyz12, 0.0),
                      axis=1, keepdims=True)
        cx = c12[0:B]
        cy = c12[B:2 * B]
        cz = c12[2 * B:3 * B]
        c3 = jnp.concatenate([cx, cy, cz], axis=1)
        nxc = jnp.where(lp_c == i, c3[:, :, None], nxc)
        dx = x - cx
        dy = y - cy
        dz = z - cz
        d = (dx * dx + dy * dy) + dz * dz
        dists = jnp.minimum(dists, d)
        m = jnp.max(dists, axis=1, keepdims=True)
        far = jnp.min(jnp.where(dists == m, lane, N), axis=1, keepdims=True)
        return dists, far, nxc

    dists0 = jnp.full((B, N), 1e10, jnp.float32)
    far0 = jnp.zeros((B, 1), jnp.int32)
    nxc0 = jnp.zeros((B, 3, NPOINT), jnp.float32)
    _, _, nxc = lax.fori_loop(0, NPOINT, step, (dists0, far0, nxc0))
    nxc_ref[...] = nxc


def _run_fps(xyz):
    xyz12 = jnp.transpose(xyz, (1, 0, 2)).reshape(3 * B, N)
    nxc = pl.pallas_call(
        _fps_body,
        out_shape=jax.ShapeDtypeStruct((B, 3, NPOINT), jnp.float32),
    )(xyz12)
    return nxc, jnp.transpose(nxc, (0, 2, 1))


def _knn_body(nxr_ref, xyz_ref, idx_ref):
    b = pl.program_id(0)
    q = nxr_ref[0]
    a = xyz_ref[0]
    x = a[0:1, :]
    y = a[1:2, :]
    z = a[2:3, :]
    dx = q[:, 0:1] - x
    dy = q[:, 1:2] - y
    dz = q[:, 2:3] - z
    d2 = (dx * dx + dy * dy) + dz * dz
    lane = lax.broadcasted_iota(jnp.int32, (TQ, N), 1)
    base = b * N
    ams = []
    for k in range(K):
        m = jnp.min(d2, axis=1, keepdims=True)
        am = jnp.min(jnp.where(d2 == m, lane, N), axis=1, keepdims=True)
        ams.append(am)
        d2 = jnp.where(lane == am, jnp.float32(jnp.inf), d2)
    idx_ref[0] = jnp.concatenate(ams, axis=1) + base


def _run_knn(nxr, xyz):
    return pl.pallas_call(
        _knn_body,
        grid=(B, NPOINT // TQ),
        in_specs=[
            pl.BlockSpec((1, TQ, 3), lambda b, t: (b, t, 0)),
            pl.BlockSpec((1, 3, N), lambda b, t: (b, 0, 0)),
        ],
        out_specs=pl.BlockSpec((1, TQ, K), lambda b, t: (b, t, 0)),
        out_shape=jax.ShapeDtypeStruct((B, NPOINT, K), jnp.int32),
    )(nxr, xyz)


NT = 512


def _format_body(xyz_ref, pts_ref, tab_ref):
    x3 = xyz_ref[0]
    p = pts_ref[0]
    pad = jnp.zeros((NT, CP - 3 - IN_DIM), jnp.float32)
    tab_ref[...] = jnp.concatenate([x3.T, p.T, pad], axis=1)


def _run_format(xyz, points):
    return pl.pallas_call(
        _format_body,
        grid=(B, N // NT),
        in_specs=[
            pl.BlockSpec((1, 3, NT), lambda b, t: (b, 0, t)),
            pl.BlockSpec((1, IN_DIM, NT), lambda b, t: (b, 0, t)),
        ],
        out_specs=pl.BlockSpec(
            (NT, CP), lambda b, t: (b * (N // NT) + t, 0)),
        out_shape=jax.ShapeDtypeStruct((B * N, CP), jnp.float32),
    )(xyz, points)


def _sc_gather_body(table_hbm, idx_hbm, out_hbm, idx_v, buf0, buf1, sem0, sem1):
    c = lax.axis_index("c")
    s = lax.axis_index("s")
    wid = s * 2 + c
    pltpu.sync_copy(idx_hbm.at[pl.ds(wid * NCH, NCH)], idx_v)
    bufs = (buf0, buf1)
    sems = (sem0, sem1)
    descs = [None, None]
    descs[0] = pltpu.async_copy(table_hbm.at[idx_v.at[0]], buf0, sem0)
    for j in range(NCH):
        cur = j % 2
        if j + 1 < NCH:
            nxt = (j + 1) % 2
            descs[nxt] = pltpu.async_copy(
                table_hbm.at[idx_v.at[j + 1]], bufs[nxt], sems[nxt])
        descs[cur].wait()
        pltpu.sync_copy(bufs[cur], out_hbm.at[pl.ds(wid * RPW + j * CH, CH)])


@functools.cache
def _sc_gather_call():
    mesh = plsc.VectorSubcoreMesh(
        core_axis_name="c", subcore_axis_name="s", num_cores=2, num_subcores=16)
    return pl.kernel(
        _sc_gather_body,
        out_type=jax.ShapeDtypeStruct((ROWS, CP), jnp.float32),
        mesh=mesh,
        scratch_types=[
            pltpu.VMEM((NCH, CH), jnp.int32),
            pltpu.VMEM((CH, CP), jnp.float32),
            pltpu.VMEM((CH, CP), jnp.float32),
            pltpu.SemaphoreType.DMA,
            pltpu.SemaphoreType.DMA,
        ],
        compiler_params=pltpu.CompilerParams(use_tc_tiling_on_sc=False),
    )


def _gather_rows(table, idxf):
    return _sc_gather_call()(table, idxf.reshape(ROWS // CH, CH))


def _passA_body(g_ref, nx_ref, w1_ref, w1x_ref, b1_ref, y1_ref, gn_ref, st_ref):
    i = pl.program_id(0)
    g = g_ref[...]
    nx = nx_ref[...]
    nxe = jnp.broadcast_to(nx[:, None, :], (QB, K, 3)).reshape(R, 3)
    gn = g[:, 0:3] - nxe
    gn_ref[0] = gn.T.reshape(3, QB, K)
    y = jnp.dot(g, w1_ref[...], preferred_element_type=jnp.float32)
    corr = jnp.dot(nx, w1x_ref[...], preferred_element_type=jnp.float32)
    corre = jnp.broadcast_to(
        corr[:, None, :], (QB, K, IN_DIM)).reshape(R, IN_DIM)
    y = y - corre + b1_ref[...]
    y1_ref[...] = y

    @pl.when(i == 0)
    def _():
        st_ref[...] = jnp.zeros_like(st_ref)

    st_ref[0:1, :] += jnp.sum(y, axis=0, keepdims=True)
    st_ref[1:2, :] += jnp.sum(y * y, axis=0, keepdims=True)


def _run_passA(g, nxq, w1t, w1xt, b1r):
    return pl.pallas_call(
        _passA_body,
        grid=(ROWS // R,),
        in_specs=[
            pl.BlockSpec((R, CP), lambda i: (i, 0)),
            pl.BlockSpec((QB, 3), lambda i: (i, 0)),
            pl.BlockSpec((CP, IN_DIM), lambda i: (0, 0)),
            pl.BlockSpec((3, IN_DIM), lambda i: (0, 0)),
            pl.BlockSpec((1, IN_DIM), lambda i: (0, 0)),
        ],
        out_specs=(
            pl.BlockSpec((R, IN_DIM), lambda i: (i, 0)),
            pl.BlockSpec((1, 3, QB, K), lambda i: (i // _TPB, 0, i % _TPB, 0)),
            pl.BlockSpec((8, IN_DIM), lambda i: (0, 0)),
        ),
        out_shape=(
            jax.ShapeDtypeStruct((ROWS, IN_DIM), jnp.float32),
            jax.ShapeDtypeStruct((B, 3, NPOINT, K), jnp.float32),
            jax.ShapeDtypeStruct((8, IN_DIM), jnp.float32),
        ),
    )(g, nxq, w1t, w1xt, b1r)


def _bn_coeffs(st_ref, gamma_ref, beta_ref):
    mean = st_ref[0:1, :] * (1.0 / ROWS)
    var = st_ref[1:2, :] * (1.0 / ROWS) - mean * mean
    a = gamma_ref[...] * lax.rsqrt(var + EPS)
    c = beta_ref[...] - mean * a
    return a, c


def _passB_body(y1_ref, st_ref, g1_ref, be1_ref, w2_ref, b2_ref, y2_ref, st2_ref):
    i = pl.program_id(0)
    a, c = _bn_coeffs(st_ref, g1_ref, be1_ref)
    h = jnp.maximum(y1_ref[...] * a + c, 0.0)
    y = jnp.dot(h, w2_ref[...], preferred_element_type=jnp.float32)
    y = y + b2_ref[...]
    y2_ref[...] = y

    @pl.when(i == 0)
    def _():
        st2_ref[...] = jnp.zeros_like(st2_ref)

    st2_ref[0:1, :] += jnp.sum(y, axis=0, keepdims=True)
    st2_ref[1:2, :] += jnp.sum(y * y, axis=0, keepdims=True)


def _run_passB(y1, st1, g1r, be1r, w2t, b2r):
    return pl.pallas_call(
        _passB_body,
        grid=(ROWS // R,),
        in_specs=[
            pl.BlockSpec((R, IN_DIM), lambda i: (i, 0)),
            pl.BlockSpec((8, IN_DIM), lambda i: (0, 0)),
            pl.BlockSpec((1, IN_DIM), lambda i: (0, 0)),
            pl.BlockSpec((1, IN_DIM), lambda i: (0, 0)),
            pl.BlockSpec((IN_DIM, OUT_DIM), lambda i: (0, 0)),
            pl.BlockSpec((1, OUT_DIM), lambda i: (0, 0)),
        ],
        out_specs=(
            pl.BlockSpec((R, OUT_DIM), lambda i: (i, 0)),
            pl.BlockSpec((8, OUT_DIM), lambda i: (0, 0)),
        ),
        out_shape=(
            jax.ShapeDtypeStruct((ROWS, OUT_DIM), jnp.float32),
            jax.ShapeDtypeStruct((8, OUT_DIM), jnp.float32),
        ),
    )(y1, st1, g1r, be1r, w2t, b2r)


def _passC_body(y2_ref, st_ref, g2_ref, be2_ref, np_ref, pool_ref):
    a, c = _bn_coeffs(st_ref, g2_ref, be2_ref)
    z = jnp.maximum(y2_ref[...] * a + c, 0.0)
    np_ref[0] = z.T
    pool_ref[0] = jnp.max(z.reshape(QB, K, OUT_DIM), axis=1).T


def _run_passC(y2, st2, g2r, be2r):
    return pl.pallas_call(
        _passC_body,
        grid=(ROWS // R,),
        in_specs=[
            pl.BlockSpec((R, OUT_DIM), lambda i: (i, 0)),
            pl.BlockSpec((8, OUT_DIM), lambda i: (0, 0)),
            pl.BlockSpec((1, OUT_DIM), lambda i: (0, 0)),
            pl.BlockSpec((1, OUT_DIM), lambda i: (0, 0)),
        ],
        out_specs=(
            pl.BlockSpec((1, OUT_DIM, R), lambda i: (i // _TPB, 0, i % _TPB)),
            pl.BlockSpec((1, OUT_DIM, QB), lambda i: (i // _TPB, 0, i % _TPB)),
        ),
        out_shape=(
            jax.ShapeDtypeStruct((B, OUT_DIM, NPOINT * K), jnp.float32),
            jax.ShapeDtypeStruct((B, OUT_DIM, NPOINT), jnp.float32),
        ),
    )(y2, st2, g2r, be2r)


def kernel(xyz, points, W1, b1, gamma1, beta1, W2, b2, gamma2, beta2):
    table = _run_format(xyz, points)

    nxc, nxr = _run_fps(xyz)
    idxf = _run_knn(nxr, xyz)
    g = _gather_rows(table, idxf.reshape(-1))

    nxq = nxr.reshape(B * NPOINT, 3)
    w1t = jnp.pad(W1.T, ((0, CP - (IN_DIM + 3)), (0, 0)))
    w1xt = W1[:, 0:3].T
    y1, gn, st1 = _run_passA(
        g, nxq, w1t, w1xt, b1.reshape(1, IN_DIM))

    y2, st2 = _run_passB(
        y1, st1, gamma1.reshape(1, IN_DIM), beta1.reshape(1, IN_DIM),
        W2.T, b2.reshape(1, OUT_DIM))

    np3, pool = _run_passC(
        y2, st2, gamma2.reshape(1, OUT_DIM), beta2.reshape(1, OUT_DIM))

    return (nxc, pool, gn, np3.reshape(B, OUT_DIM, NPOINT, K))
